# Initial kernel scaffold; baseline (speedup 1.0000x reference)
import jax
import jax.numpy as jnp
from jax.experimental import pallas as pl


def kernel(input_ids, token_type_ids, word_table, pos_table, type_table, ln_weight, ln_bias):
    raise NotImplementedError("write your pallas kernel here")


# speedup vs baseline: 2.7756x; 2.7756x over previous
import functools

import jax
import jax.numpy as jnp
from jax import lax
from jax.experimental import pallas as pl
from jax.experimental.pallas import tpu as pltpu
from jax.experimental.pallas import tpu_sc as plsc

VOCAB = 100000
HIDDEN = 128
MAX_POS = 512
EPS = 1e-12
B, S = 1024, 200
N = B * S
NC, NS, L = 2, 16, 16
NW = NC * NS
SEQ_PER_W = B // NW
KH = HIDDEN // L

_MAGIC = 0x5F3759DF


def _rsqrt16(v):
  iv = lax.bitcast_convert_type(v, jnp.int32)
  y = lax.bitcast_convert_type(jnp.int32(_MAGIC) - (iv >> 1), jnp.float32)
  half_v = 0.5 * v
  for _ in range(3):
    y = y * (1.5 - half_v * y * y)
  return y


def _sc_kernel(ids_hbm, tt_hbm, word_hbm, pos_hbm, type_hbm, w_hbm, b_hbm,
               out_hbm, idx_a, idx_b, tt_v, rows_v, pos_v, type_v, wb_v, sem):
  wid = lax.axis_index("s") * NC + lax.axis_index("c")

  pltpu.sync_copy(pos_hbm.at[pl.ds(0, S)], pos_v)
  pltpu.sync_copy(type_hbm, type_v)
  pltpu.sync_copy(w_hbm, wb_v.at[0])
  pltpu.sync_copy(b_hbm, wb_v.at[1])

  t0 = [type_v[0, pl.ds(k * L, L)] for k in range(KH)]
  td = [type_v[1, pl.ds(k * L, L)] - t0[k] for k in range(KH)]
  wv = [wb_v[0, pl.ds(k * L, L)] for k in range(KH)]
  bv = [wb_v[1, pl.ds(k * L, L)] for k in range(KH)]

  def fold_body(i, _):
    for k in range(KH):
      pos_v[i, pl.ds(k * L, L)] = pos_v[i, pl.ds(k * L, L)] + t0[k]
    return 0
  lax.fori_loop(0, S, fold_body, 0)

  inv_h = jnp.float32(1.0 / HIDDEN)

  def seq_body(j, _):
    base = (wid * SEQ_PER_W + j) * S
    pltpu.sync_copy(ids_hbm.at[pl.ds(base, 128)], idx_a)
    pltpu.sync_copy(ids_hbm.at[pl.ds(base + 128, S - 128)], idx_b)
    pltpu.sync_copy(tt_hbm.at[pl.ds(base, S)], tt_v.at[pl.ds(0, S)])
    pltpu.async_copy(word_hbm.at[idx_a], rows_v.at[pl.ds(0, 128)], sem).wait()
    pltpu.async_copy(word_hbm.at[idx_b], rows_v.at[pl.ds(128, S - 128)],
                     sem).wait()

    def tok_body(i, _):
      t = tt_v[pl.ds(i, L)][0]
      tf = jnp.broadcast_to(t, (L,)).astype(jnp.float32)
      xs = []
      s = jnp.zeros((L,), jnp.float32)
      q = jnp.zeros((L,), jnp.float32)
      for k in range(KH):
        x = (rows_v[i, pl.ds(k * L, L)] + pos_v[i, pl.ds(k * L, L)]
             + tf * td[k])
        xs.append(x)
        s = s + x
        q = q + x * x
      mean = jnp.sum(s) * inv_h
      var = jnp.maximum(jnp.sum(q) * inv_h - mean * mean, 0.0) + EPS
      meanv = jnp.broadcast_to(mean, (L,))
      invv = _rsqrt16(jnp.broadcast_to(var, (L,)))
      for k in range(KH):
        rows_v[i, pl.ds(k * L, L)] = (xs[k] - meanv) * invv * wv[k] + bv[k]
      return 0
    lax.fori_loop(0, S, tok_body, 0)

    pltpu.sync_copy(rows_v, out_hbm.at[pl.ds(base, S)])
    return 0

  lax.fori_loop(0, SEQ_PER_W, seq_body, 0)


@jax.jit
def kernel(input_ids, token_type_ids, word_table, pos_table, type_table,
           ln_weight, ln_bias):
  ids = input_ids.reshape(N)
  tts = token_type_ids.reshape(N)
  mesh = plsc.VectorSubcoreMesh(core_axis_name="c", subcore_axis_name="s")
  run = functools.partial(
      pl.kernel,
      out_type=jax.ShapeDtypeStruct((N, HIDDEN), jnp.float32),
      mesh=mesh,
      compiler_params=pltpu.CompilerParams(needs_layout_passes=False),
      scratch_types=[
          pltpu.VMEM((128,), jnp.int32),
          pltpu.VMEM((S - 128,), jnp.int32),
          pltpu.VMEM((S + L,), jnp.int32),
          pltpu.VMEM((S, HIDDEN), jnp.float32),
          pltpu.VMEM((S, HIDDEN), jnp.float32),
          pltpu.VMEM((2, HIDDEN), jnp.float32),
          pltpu.VMEM((2, HIDDEN), jnp.float32),
          pltpu.SemaphoreType.DMA,
      ],
  )(_sc_kernel)
  out = run(ids, tts, word_table, pos_table, type_table, ln_weight, ln_bias)
  return out.reshape(B, S, HIDDEN)


# speedup vs baseline: 5.1241x; 1.8461x over previous
import functools

import jax
import jax.numpy as jnp
from jax import lax
from jax.experimental import pallas as pl
from jax.experimental.pallas import tpu as pltpu
from jax.experimental.pallas import tpu_sc as plsc

HIDDEN = 128
EPS = 1e-12
B, S = 1024, 200
N = B * S
NC, NS, L = 2, 16, 16
NW = NC * NS
TPW = N // NW
CHUNK = 64
NCHUNK = TPW // CHUNK
NBUF = 4
KH = HIDDEN // L

_MAGIC = 0x5F3759DF


def _rsqrt16(v):
  iv = lax.bitcast_convert_type(v, jnp.int32)
  y = lax.bitcast_convert_type(jnp.int32(_MAGIC) - (iv >> 1), jnp.float32)
  half_v = 0.5 * v
  for _ in range(3):
    y = y * (1.5 - half_v * y * y)
  return y


def _sc_kernel(ids_hbm, tt_hbm, word_hbm, pt_hbm, w_hbm, b_hbm, out_hbm,
               ids_v, tt_v, rows_v, pt_v, wb_v, sem_g, sem_w):
  wid = lax.axis_index("s") * NC + lax.axis_index("c")
  wbase = wid * TPW

  pltpu.sync_copy(pt_hbm, pt_v)
  pltpu.sync_copy(w_hbm, wb_v.at[0])
  pltpu.sync_copy(b_hbm, wb_v.at[1])
  pltpu.sync_copy(ids_hbm.at[pl.ds(wbase, TPW)], ids_v)
  pltpu.sync_copy(tt_hbm.at[pl.ds(wbase, TPW)], tt_v.at[pl.ds(0, TPW)])

  wv = [wb_v[0, pl.ds(k * L, L)] for k in range(KH)]
  bv = [wb_v[1, pl.ds(k * L, L)] for k in range(KH)]
  inv_h = jnp.float32(1.0 / HIDDEN)

  def gather_start(g, b):
    start = pl.multiple_of(g * CHUNK, 64)
    return pltpu.async_copy(
        word_hbm.at[ids_v.at[pl.ds(start, CHUNK)]],
        rows_v.at[b], sem_g.at[b])

  for g0 in range(NBUF - 1):
    gather_start(g0, g0)

  def chunk_body(g, _):
    b = lax.rem(g, NBUF)
    start = pl.multiple_of(g * CHUNK, 64)
    pltpu.make_async_copy(
        word_hbm.at[ids_v.at[pl.ds(start, CHUNK)]],
        rows_v.at[b], sem_g.at[b]).wait()

    def tok_body(i):
      l = g * CHUNK + i
      tvec = tt_v[pl.ds(l, L)]
      r = lax.rem(l, S) + tvec[0] * S
      x = [rows_v[b, i, pl.ds(k * L, L)] + pt_v[r, pl.ds(k * L, L)]
           for k in range(KH)]
      s01, s23 = x[0] + x[1], x[2] + x[3]
      s45, s67 = x[4] + x[5], x[6] + x[7]
      s = (s01 + s23) + (s45 + s67)
      q01, q23 = x[0] * x[0] + x[1] * x[1], x[2] * x[2] + x[3] * x[3]
      q45, q67 = x[4] * x[4] + x[5] * x[5], x[6] * x[6] + x[7] * x[7]
      q = (q01 + q23) + (q45 + q67)
      mean = jnp.sum(s) * inv_h
      var = jnp.maximum(jnp.sum(q) * inv_h - mean * mean, 0.0) + EPS
      meanv = jnp.broadcast_to(mean, (L,))
      invv = _rsqrt16(jnp.broadcast_to(var, (L,)))
      for k in range(KH):
        rows_v[b, i, pl.ds(k * L, L)] = (x[k] - meanv) * invv * wv[k] + bv[k]

    plsc.parallel_loop(0, CHUNK, 1, unroll=4)(tok_body)

    out_start = pl.multiple_of(wbase + g * CHUNK, 64)
    pltpu.async_copy(rows_v.at[b], out_hbm.at[pl.ds(out_start, CHUNK)],
                     sem_w.at[b])

    nxt = g + NBUF - 1
    b3 = lax.rem(nxt, NBUF)

    @pl.when(nxt < NCHUNK)
    def _():
      @pl.when(g >= 1)
      def _():
        prev = nxt - NBUF
        prev_start = pl.multiple_of(wbase + prev * CHUNK, 64)
        pltpu.make_async_copy(
            rows_v.at[b3], out_hbm.at[pl.ds(prev_start, CHUNK)],
            sem_w.at[b3]).wait()
      gather_start(nxt, b3)

    return 0

  lax.fori_loop(0, NCHUNK, chunk_body, 0)

  for b in range(NBUF):
    g = NCHUNK - NBUF + b
    start = pl.multiple_of(wbase + g * CHUNK, 64)
    pltpu.make_async_copy(rows_v.at[b], out_hbm.at[pl.ds(start, CHUNK)],
                          sem_w.at[b]).wait()


@jax.jit
def kernel(input_ids, token_type_ids, word_table, pos_table, type_table,
           ln_weight, ln_bias):
  ids = input_ids.reshape(N)
  tts = token_type_ids.reshape(N)
  pt = (type_table[:, None, :] + pos_table[None, :S, :]).reshape(2 * S, HIDDEN)
  mesh = plsc.VectorSubcoreMesh(core_axis_name="c", subcore_axis_name="s")
  run = functools.partial(
      pl.kernel,
      out_type=jax.ShapeDtypeStruct((N, HIDDEN), jnp.float32),
      mesh=mesh,
      compiler_params=pltpu.CompilerParams(needs_layout_passes=False),
      scratch_types=[
          pltpu.VMEM((TPW,), jnp.int32),
          pltpu.VMEM((TPW + L,), jnp.int32),
          pltpu.VMEM((NBUF, CHUNK, HIDDEN), jnp.float32),
          pltpu.VMEM((2 * S, HIDDEN), jnp.float32),
          pltpu.VMEM((2, HIDDEN), jnp.float32),
          pltpu.SemaphoreType.DMA((NBUF,)),
          pltpu.SemaphoreType.DMA((NBUF,)),
      ],
  )(_sc_kernel)
  out = run(ids, tts, word_table, pt, ln_weight, ln_bias)
  return out.reshape(B, S, HIDDEN)
